# baseline (speedup 1.0000x reference)
import functools

import jax
import jax.numpy as jnp
from jax import lax
from jax.experimental import pallas as pl
from jax.experimental.pallas import tpu as pltpu
from jax.experimental.pallas import tpu_sc as plsc

N = 50000
E = 800000
H = 64
G = 64
NCLS = 6

EPAD = 819200
EROWS = EPAD // 128
NACC = 51200
NC = 2
NS = 16

BLK = 1000
NB = N // BLK

@functools.cache
def _mesh():
    return plsc.VectorSubcoreMesh(core_axis_name="c", subcore_axis_name="s",
                                  num_cores=NC, num_subcores=NS)


def _sc_degree(dst_rows, ones8, zeros8):
    rows_per_worker = EROWS // (NC * NS)
    sup = rows_per_worker // 8
    zrows = NACC // NS

    @functools.partial(
        pl.kernel,
        out_type=jax.ShapeDtypeStruct((2, N, 8), jnp.float32),
        mesh=_mesh(),
        compiler_params=pltpu.CompilerParams(use_tc_tiling_on_sc=False),
        scratch_types=[
            pltpu.VMEM((8, 128), jnp.int32),
            pltpu.VMEM((128, 8), jnp.float32),
            pltpu.VMEM((128, 8), jnp.float32),
            pltpu.MemorySpace.VMEM_SHARED((NACC, 8), jnp.float32),
            pltpu.SemaphoreType.DMA,
        ],
    )
    def k(dst_ref, ones_ref, z_ref, out_ref, dbuf, onebuf, zbuf, acc, sem):
        cid = lax.axis_index("c")
        sid = lax.axis_index("s")
        wid = sid * NC + cid
        pltpu.sync_copy(ones_ref, onebuf)
        pltpu.sync_copy(z_ref, zbuf)

        z0 = sid * zrows
        def zero_body(i, _):
            pltpu.sync_copy(zbuf, acc.at[pl.ds(pl.multiple_of(z0 + i * 128, 8), 128)])
            return 0
        lax.fori_loop(0, zrows // 128, zero_body, 0)
        plsc.subcore_barrier()

        row0 = wid * rows_per_worker
        def chunk(g, _):
            r0 = pl.multiple_of(row0 + g * 8, 8)
            pltpu.sync_copy(dst_ref.at[pl.ds(r0, 8)], dbuf)
            for j in range(8):
                pltpu.sync_copy(onebuf, acc.at[dbuf.at[j]], add=True)
            return 0
        lax.fori_loop(0, sup, chunk, 0)
        plsc.subcore_barrier()

        w0 = sid * zrows
        @pl.when(sid < NS - 1)
        def _():
            pltpu.sync_copy(acc.at[pl.ds(pl.multiple_of(w0, 8), zrows)],
                            out_ref.at[cid, pl.ds(pl.multiple_of(w0, 8), zrows)])
        @pl.when(sid == NS - 1)
        def _():
            pltpu.sync_copy(acc.at[pl.ds(N - zrows, zrows)],
                            out_ref.at[cid, pl.ds(N - zrows, zrows)])

    return k(dst_rows, ones8, zeros8)


def _sc_aggregate(src_rows, dst_rows, u0, u1, zeros32):
    rows_per_sub = EROWS // NS
    nsuper = rows_per_sub // 8
    npair = nsuper // 2
    zrows = NACC // NS

    @functools.partial(
        pl.kernel,
        out_type=jax.ShapeDtypeStruct((2, N, 32), jnp.float32),
        mesh=_mesh(),
        compiler_params=pltpu.CompilerParams(use_tc_tiling_on_sc=False),
        scratch_types=[
            pltpu.VMEM((8, 128), jnp.int32),
            pltpu.VMEM((8, 128), jnp.int32),
            pltpu.VMEM((8, 128), jnp.int32),
            pltpu.VMEM((8, 128), jnp.int32),
            pltpu.VMEM((256, 32), jnp.float32),
            pltpu.VMEM((256, 32), jnp.float32),
            pltpu.MemorySpace.VMEM_SHARED((NACC, 32), jnp.float32),
            pltpu.SemaphoreType.DMA,
            pltpu.SemaphoreType.DMA,
        ],
    )
    def k(src_ref, dst_ref, u0_ref, u1_ref, z_ref, out_ref,
          sx0, dx0, sx1, dx1, rows0, rows1, acc, sem0, sem1):
        cid = lax.axis_index("c")
        sid = lax.axis_index("s")

        z0 = sid * zrows
        for i in range(zrows // 128):
            pltpu.async_copy(
                z_ref, acc.at[pl.ds(pl.multiple_of(z0 + i * 128, 8), 128)],
                sem0)
        def zdrain(i, _):
            pltpu.make_async_copy(z_ref,
                                  acc.at[pl.ds(pl.multiple_of(z0, 8), 128)],
                                  sem0).wait()
            return 0
        lax.fori_loop(0, zrows // 128, zdrain, 0)
        plsc.subcore_barrier()

        row0 = sid * rows_per_sub
        rbufs = (rows0, rows1)
        sems = (sem0, sem1)

        def run(uref):
            def load_super(s, sb, db):
                r0 = pl.multiple_of(row0 + s * 8, 8)
                pltpu.sync_copy(src_ref.at[pl.ds(r0, 8)], sb)
                pltpu.sync_copy(dst_ref.at[pl.ds(r0, 8)], db)

            def fire2(sb, j0, rows, sem):
                for j in (j0, j0 + 1):
                    pltpu.async_copy(uref.at[sb.at[j]],
                                     rows.at[pl.ds((j - j0) * 128, 128)], sem)

            def drain2(rows, sem):
                pltpu.make_async_copy(uref.at[pl.ds(0, 256)], rows, sem).wait()

            def scat2(db, j0, rows):
                for j in (j0, j0 + 1):
                    pltpu.sync_copy(rows.at[pl.ds((j - j0) * 128, 128)],
                                    acc.at[db.at[j]], add=True)

            load_super(0, sx0, dx0)
            fire2(sx0, 0, rows0, sem0)

            def pair(p, _):
                load_super(2 * p + 1, sx1, dx1)
                for c in range(8):
                    sb, db = (sx0, dx0) if c < 4 else (sx1, dx1)
                    j0 = (c % 4) * 2
                    cur, nxt = c % 2, (c + 1) % 2
                    if c < 7:
                        nsb = sx0 if c + 1 < 4 else sx1
                        fire2(nsb, ((c + 1) % 4) * 2, rbufs[nxt], sems[nxt])
                    else:
                        @pl.when(p < npair - 1)
                        def _():
                            load_super(2 * p + 2, sx0, dx0)
                            fire2(sx0, 0, rbufs[nxt], sems[nxt])
                    drain2(rbufs[cur], sems[cur])
                    scat2(db, j0, rbufs[cur])
                return 0
            lax.fori_loop(0, npair, pair, 0)

        @pl.when(cid == 0)
        def _():
            run(u0_ref)
        @pl.when(cid == 1)
        def _():
            run(u1_ref)

        plsc.subcore_barrier()
        w0 = sid * zrows
        @pl.when(sid < NS - 1)
        def _():
            pltpu.sync_copy(acc.at[pl.ds(pl.multiple_of(w0, 8), zrows)],
                            out_ref.at[cid, pl.ds(pl.multiple_of(w0, 8), zrows)])
        @pl.when(sid == NS - 1)
        def _():
            pltpu.sync_copy(acc.at[pl.ds(N - zrows, zrows)],
                            out_ref.at[cid, pl.ds(N - zrows, zrows)])

    return k(src_rows, dst_rows, u0, u1, zeros32)


def _sc_segmax(h, batch):
    per_w = 1568
    cs_sz = 112
    nchunks = per_w // cs_sz

    @functools.partial(
        pl.kernel,
        out_type=jax.ShapeDtypeStruct((NC * NS, G, H), jnp.float32),
        mesh=_mesh(),
        compiler_params=pltpu.CompilerParams(use_tc_tiling_on_sc=False),
        scratch_types=[
            pltpu.VMEM((cs_sz, H), jnp.float32),
            pltpu.VMEM((cs_sz + 16,), jnp.int32),
            pltpu.VMEM((G, H), jnp.float32),
        ],
    )
    def k(h_ref, b_ref, out_ref, hbuf, bbuf, table):
        cid = lax.axis_index("c")
        sid = lax.axis_index("s")
        wid = sid * NC + cid
        neg = jnp.full((16,), -jnp.inf, jnp.float32)

        def init_row(i, _):
            for kk in range(H // 16):
                table[i, pl.ds(kk * 16, 16)] = neg
            return 0
        lax.fori_loop(0, G, init_row, 0)

        start = wid * per_w

        def chunk(j, _):
            cs = jnp.minimum(start + j * cs_sz, N - cs_sz)
            cs = pl.multiple_of(cs, 8)
            pltpu.sync_copy(b_ref.at[pl.ds(cs, cs_sz)], bbuf.at[pl.ds(0, cs_sz)])
            pltpu.sync_copy(h_ref.at[pl.ds(cs, cs_sz)], hbuf)

            def node(i, _):
                g = bbuf[pl.ds(i, 16)][0]
                for kk in range(H // 16):
                    t = table[g, pl.ds(kk * 16, 16)]
                    r = hbuf[i, pl.ds(kk * 16, 16)]
                    table[g, pl.ds(kk * 16, 16)] = jnp.maximum(t, r)
                return 0
            lax.fori_loop(0, cs_sz, node, 0)
            return 0
        lax.fori_loop(0, nchunks, chunk, 0)

        pltpu.sync_copy(table, out_ref.at[wid])

    return k(h, batch)


def _tc_pre1(x, cnt, W1):
    def body(x_ref, cnt_ref, w_ref, u0_ref, u1_ref, dinv_ref):
        deg = cnt_ref[0, :, 0:1] + cnt_ref[1, :, 0:1] + 1.0
        dinv = lax.rsqrt(deg)
        u = jnp.dot(x_ref[...], w_ref[...],
                    preferred_element_type=jnp.float32) * dinv
        u0_ref[...] = u[:, :32]
        u1_ref[...] = u[:, 32:]
        dinv_ref[...] = dinv

    return pl.pallas_call(
        body,
        grid=(NB,),
        in_specs=[
            pl.BlockSpec((BLK, H), lambda i: (i, 0)),
            pl.BlockSpec((2, BLK, 8), lambda i: (0, i, 0)),
            pl.BlockSpec((H, H), lambda i: (0, 0)),
        ],
        out_specs=[
            pl.BlockSpec((BLK, 32), lambda i: (i, 0)),
            pl.BlockSpec((BLK, 32), lambda i: (i, 0)),
            pl.BlockSpec((BLK, 1), lambda i: (i, 0)),
        ],
        out_shape=[
            jax.ShapeDtypeStruct((N, 32), jnp.float32),
            jax.ShapeDtypeStruct((N, 32), jnp.float32),
            jax.ShapeDtypeStruct((N, 1), jnp.float32),
        ],
    )(x, cnt, W1)


def _tc_post(acc, u0, u1, dinv, b):
    def body(acc_ref, u0_ref, u1_ref, dinv_ref, b_ref, y_ref, st_ref):
        i = pl.program_id(0)
        a = jnp.concatenate([acc_ref[0], acc_ref[1]], axis=1)
        u = jnp.concatenate([u0_ref[...], u1_ref[...]], axis=1)
        y = (a + u) * dinv_ref[...] + b_ref[...]
        y_ref[...] = y
        s0 = jnp.sum(y, axis=0, keepdims=True)
        s1 = jnp.sum(y * y, axis=0, keepdims=True)
        st = jnp.concatenate([s0, s1], axis=0)
        @pl.when(i == 0)
        def _():
            st_ref[...] = st
        @pl.when(i > 0)
        def _():
            st_ref[...] += st

    return pl.pallas_call(
        body,
        grid=(NB,),
        in_specs=[
            pl.BlockSpec((2, BLK, 32), lambda i: (0, i, 0)),
            pl.BlockSpec((BLK, 32), lambda i: (i, 0)),
            pl.BlockSpec((BLK, 32), lambda i: (i, 0)),
            pl.BlockSpec((BLK, 1), lambda i: (i, 0)),
            pl.BlockSpec((1, H), lambda i: (0, 0)),
        ],
        out_specs=[
            pl.BlockSpec((BLK, H), lambda i: (i, 0)),
            pl.BlockSpec((2, H), lambda i: (0, 0)),
        ],
        out_shape=[
            jax.ShapeDtypeStruct((N, H), jnp.float32),
            jax.ShapeDtypeStruct((2, H), jnp.float32),
        ],
    )(acc, u0, u1, dinv, b)


def _tc_pre(y, st, gam, bet, W, dinv):
    def body(y_ref, st_ref, g_ref, be_ref, w_ref, dinv_ref, u0_ref, u1_ref):
        m = st_ref[0:1, :] * (1.0 / N)
        v = st_ref[1:2, :] * (1.0 / N) - m * m
        hh = jax.nn.relu((y_ref[...] - m) * lax.rsqrt(v + 1e-5)
                         * g_ref[...] + be_ref[...])
        u = jnp.dot(hh, w_ref[...],
                    preferred_element_type=jnp.float32) * dinv_ref[...]
        u0_ref[...] = u[:, :32]
        u1_ref[...] = u[:, 32:]

    return pl.pallas_call(
        body,
        grid=(NB,),
        in_specs=[
            pl.BlockSpec((BLK, H), lambda i: (i, 0)),
            pl.BlockSpec((2, H), lambda i: (0, 0)),
            pl.BlockSpec((1, H), lambda i: (0, 0)),
            pl.BlockSpec((1, H), lambda i: (0, 0)),
            pl.BlockSpec((H, H), lambda i: (0, 0)),
            pl.BlockSpec((BLK, 1), lambda i: (i, 0)),
        ],
        out_specs=[
            pl.BlockSpec((BLK, 32), lambda i: (i, 0)),
            pl.BlockSpec((BLK, 32), lambda i: (i, 0)),
        ],
        out_shape=[
            jax.ShapeDtypeStruct((N, 32), jnp.float32),
            jax.ShapeDtypeStruct((N, 32), jnp.float32),
        ],
    )(y, st, gam, bet, W, dinv)


def _tc_pool(y, st, gam, bet, batch2d):
    def body(y_ref, st_ref, g_ref, be_ref, b_ref, h_ref, gs_ref, gc_ref):
        i = pl.program_id(0)
        m = st_ref[0:1, :] * (1.0 / N)
        v = st_ref[1:2, :] * (1.0 / N) - m * m
        hh = jax.nn.relu((y_ref[...] - m) * lax.rsqrt(v + 1e-5)
                         * g_ref[...] + be_ref[...])
        h_ref[...] = hh
        ids = lax.broadcasted_iota(jnp.int32, (G, BLK), 0)
        oh = (ids == b_ref[0]).astype(jnp.float32)
        gs = jnp.dot(oh, hh, preferred_element_type=jnp.float32)
        gc = jnp.sum(oh, axis=1, keepdims=True)
        @pl.when(i == 0)
        def _():
            gs_ref[...] = gs
            gc_ref[...] = gc
        @pl.when(i > 0)
        def _():
            gs_ref[...] += gs
            gc_ref[...] += gc

    return pl.pallas_call(
        body,
        grid=(NB,),
        in_specs=[
            pl.BlockSpec((BLK, H), lambda i: (i, 0)),
            pl.BlockSpec((2, H), lambda i: (0, 0)),
            pl.BlockSpec((1, H), lambda i: (0, 0)),
            pl.BlockSpec((1, H), lambda i: (0, 0)),
            pl.BlockSpec((1, 1, BLK), lambda i: (i, 0, 0)),
        ],
        out_specs=[
            pl.BlockSpec((BLK, H), lambda i: (i, 0)),
            pl.BlockSpec((G, H), lambda i: (0, 0)),
            pl.BlockSpec((G, 1), lambda i: (0, 0)),
        ],
        out_shape=[
            jax.ShapeDtypeStruct((N, H), jnp.float32),
            jax.ShapeDtypeStruct((G, H), jnp.float32),
            jax.ShapeDtypeStruct((G, 1), jnp.float32),
        ],
    )(y, st, gam, bet, batch2d)


def _tc_head(gs, gc, maxp, LW1, Lb1, LW2, Lb2, LW3, Lb3):
    def body(gs_ref, gc_ref, mp_ref, w1_ref, b1_ref, w2_ref, b2_ref,
             w3_ref, b3_ref, out_ref):
        mx = jnp.max(mp_ref[...], axis=0)
        mean = gs_ref[...] / jnp.maximum(gc_ref[...], 1.0)
        z = jnp.concatenate([mean, mx], axis=1)
        z = jax.nn.relu(jnp.dot(z, w1_ref[...],
                                preferred_element_type=jnp.float32) + b1_ref[...])
        z = jax.nn.relu(jnp.dot(z, w2_ref[...],
                                preferred_element_type=jnp.float32) + b2_ref[...])
        out_ref[...] = jnp.dot(z, w3_ref[...],
                               preferred_element_type=jnp.float32) + b3_ref[...]

    return pl.pallas_call(
        body,
        out_shape=jax.ShapeDtypeStruct((G, NCLS), jnp.float32),
    )(gs, gc, maxp, LW1, Lb1, LW2, Lb2, LW3, Lb3)


def kernel(x, edge_index, batch, W1, b1, W2, b2, W3, b3, g1, be1, g2, be2,
           g3, be3, LW1, Lb1, LW2, Lb2, LW3, Lb3):
    src = edge_index[0]
    dst = edge_index[1]
    pad = EPAD - E
    src_rows = jnp.concatenate(
        [src, jnp.zeros((pad,), jnp.int32)]).reshape(EROWS, 128)
    dst_rows = jnp.concatenate(
        [dst, jnp.full((pad,), N, jnp.int32)]).reshape(EROWS, 128)
    ones8 = jnp.zeros((128, 8), jnp.float32).at[:, 0].set(1.0)
    zeros8 = jnp.zeros((128, 8), jnp.float32)
    zeros32 = jnp.zeros((128, 32), jnp.float32)
    batch2d = batch.reshape(NB, 1, BLK)

    cnt = _sc_degree(dst_rows, ones8, zeros8)
    u0, u1, dinv = _tc_pre1(x, cnt, W1)

    acc = _sc_aggregate(src_rows, dst_rows, u0, u1, zeros32)
    y, st = _tc_post(acc, u0, u1, dinv, b1.reshape(1, H))

    u0, u1 = _tc_pre(y, st, g1.reshape(1, H), be1.reshape(1, H), W2, dinv)
    acc = _sc_aggregate(src_rows, dst_rows, u0, u1, zeros32)
    y, st = _tc_post(acc, u0, u1, dinv, b2.reshape(1, H))

    u0, u1 = _tc_pre(y, st, g2.reshape(1, H), be2.reshape(1, H), W3, dinv)
    acc = _sc_aggregate(src_rows, dst_rows, u0, u1, zeros32)
    y, st = _tc_post(acc, u0, u1, dinv, b3.reshape(1, H))

    h, gs, gc = _tc_pool(y, st, g3.reshape(1, H), be3.reshape(1, H), batch2d)
    maxp = _sc_segmax(h, batch)

    return _tc_head(gs, gc, maxp, LW1, Lb1.reshape(1, H),
                    LW2, Lb2.reshape(1, H // 2), LW3, Lb3.reshape(1, NCLS))


# speedup vs baseline: 1.0750x; 1.0750x over previous
import functools

import jax
import jax.numpy as jnp
from jax import lax
from jax.experimental import pallas as pl
from jax.experimental.pallas import tpu as pltpu
from jax.experimental.pallas import tpu_sc as plsc

N = 50000
E = 800000
H = 64
G = 64
NCLS = 6

EPAD = 819200
EROWS = EPAD // 128
NACC = 51200
NC = 2
NS = 16

BLK = 1000
NB = N // BLK

@functools.cache
def _mesh():
    return plsc.VectorSubcoreMesh(core_axis_name="c", subcore_axis_name="s",
                                  num_cores=NC, num_subcores=NS)


def _sc_degree(dst_rows, ones8, zeros8):
    rows_per_worker = EROWS // (NC * NS)
    sup = rows_per_worker // 8
    zrows = NACC // NS

    @functools.partial(
        pl.kernel,
        out_type=jax.ShapeDtypeStruct((2, N, 8), jnp.float32),
        mesh=_mesh(),
        compiler_params=pltpu.CompilerParams(use_tc_tiling_on_sc=False),
        scratch_types=[
            pltpu.VMEM((8, 128), jnp.int32),
            pltpu.VMEM((128, 8), jnp.float32),
            pltpu.VMEM((128, 8), jnp.float32),
            pltpu.MemorySpace.VMEM_SHARED((NACC, 8), jnp.float32),
            pltpu.SemaphoreType.DMA,
        ],
    )
    def k(dst_ref, ones_ref, z_ref, out_ref, dbuf, onebuf, zbuf, acc, sem):
        cid = lax.axis_index("c")
        sid = lax.axis_index("s")
        wid = sid * NC + cid
        pltpu.sync_copy(ones_ref, onebuf)
        pltpu.sync_copy(z_ref, zbuf)

        z0 = sid * zrows
        def zero_body(i, _):
            pltpu.sync_copy(zbuf, acc.at[pl.ds(pl.multiple_of(z0 + i * 128, 8), 128)])
            return 0
        lax.fori_loop(0, zrows // 128, zero_body, 0)
        plsc.subcore_barrier()

        row0 = wid * rows_per_worker
        def chunk(g, _):
            r0 = pl.multiple_of(row0 + g * 8, 8)
            pltpu.sync_copy(dst_ref.at[pl.ds(r0, 8)], dbuf)
            for j in range(8):
                pltpu.sync_copy(onebuf, acc.at[dbuf.at[j]], add=True)
            return 0
        lax.fori_loop(0, sup, chunk, 0)
        plsc.subcore_barrier()

        w0 = sid * zrows
        @pl.when(sid < NS - 1)
        def _():
            pltpu.sync_copy(acc.at[pl.ds(pl.multiple_of(w0, 8), zrows)],
                            out_ref.at[cid, pl.ds(pl.multiple_of(w0, 8), zrows)])
        @pl.when(sid == NS - 1)
        def _():
            pltpu.sync_copy(acc.at[pl.ds(N - zrows, zrows)],
                            out_ref.at[cid, pl.ds(N - zrows, zrows)])

    return k(dst_rows, ones8, zeros8)


def _sc_aggregate(src_rows, dst_rows, u0, u1, zeros32):
    rows_per_sub = EROWS // NS
    nsuper = rows_per_sub // 8
    npair = nsuper // 2
    zrows = NACC // NS

    @functools.partial(
        pl.kernel,
        out_type=jax.ShapeDtypeStruct((2, N, 32), jnp.float32),
        mesh=_mesh(),
        compiler_params=pltpu.CompilerParams(use_tc_tiling_on_sc=False),
        scratch_types=[
            pltpu.VMEM((8, 128), jnp.int32),
            pltpu.VMEM((8, 128), jnp.int32),
            pltpu.VMEM((8, 128), jnp.int32),
            pltpu.VMEM((8, 128), jnp.int32),
            pltpu.VMEM((256, 32), jnp.float32),
            pltpu.VMEM((256, 32), jnp.float32),
            pltpu.MemorySpace.VMEM_SHARED((NACC, 32), jnp.float32),
            pltpu.SemaphoreType.DMA,
            pltpu.SemaphoreType.DMA,
            pltpu.SemaphoreType.DMA,
        ],
    )
    def k(src_ref, dst_ref, u0_ref, u1_ref, z_ref, out_ref,
          sx0, dx0, sx1, dx1, rows0, rows1, acc, sem0, sem1, isem):
        cid = lax.axis_index("c")
        sid = lax.axis_index("s")
        pltpu.sync_copy(z_ref, rows0.at[pl.ds(0, 128)])

        z0 = sid * zrows
        def zero_body(i, _):
            pltpu.sync_copy(rows0.at[pl.ds(0, 128)],
                            acc.at[pl.ds(pl.multiple_of(z0 + i * 128, 8), 128)])
            return 0
        lax.fori_loop(0, zrows // 128, zero_body, 0)
        plsc.subcore_barrier()

        row0 = sid * rows_per_sub
        rbufs = (rows0, rows1)
        sems = (sem0, sem1)

        def run(uref):
            def load_super(s, sb, db):
                r0 = pl.multiple_of(row0 + s * 8, 8)
                pltpu.sync_copy(src_ref.at[pl.ds(r0, 8)], sb)
                pltpu.sync_copy(dst_ref.at[pl.ds(r0, 8)], db)

            def prefetch_super(s, sb, db):
                r0 = pl.multiple_of(row0 + s * 8, 8)
                pltpu.async_copy(src_ref.at[pl.ds(r0, 8)], sb, isem)
                pltpu.async_copy(dst_ref.at[pl.ds(r0, 8)], db, isem)

            def drain_idx(sb, db):
                pltpu.make_async_copy(src_ref.at[pl.ds(0, 8)], sb, isem).wait()
                pltpu.make_async_copy(dst_ref.at[pl.ds(0, 8)], db, isem).wait()

            def fire2(sb, j0, rows, sem):
                for j in (j0, j0 + 1):
                    pltpu.async_copy(uref.at[sb.at[j]],
                                     rows.at[pl.ds((j - j0) * 128, 128)], sem)

            def drain2(rows, sem):
                pltpu.make_async_copy(uref.at[pl.ds(0, 256)], rows, sem).wait()

            def scat2(db, j0, rows):
                for j in (j0, j0 + 1):
                    pltpu.sync_copy(rows.at[pl.ds((j - j0) * 128, 128)],
                                    acc.at[db.at[j]], add=True)

            load_super(0, sx0, dx0)
            fire2(sx0, 0, rows0, sem0)

            def pair(p, _):
                prefetch_super(2 * p + 1, sx1, dx1)
                for c in range(8):
                    sb, db = (sx0, dx0) if c < 4 else (sx1, dx1)
                    j0 = (c % 4) * 2
                    cur, nxt = c % 2, (c + 1) % 2
                    if c == 3:
                        drain_idx(sx1, dx1)
                    if c == 4:
                        @pl.when(p < npair - 1)
                        def _():
                            prefetch_super(2 * p + 2, sx0, dx0)
                    if c < 7:
                        nsb = sx0 if c + 1 < 4 else sx1
                        fire2(nsb, ((c + 1) % 4) * 2, rbufs[nxt], sems[nxt])
                    else:
                        @pl.when(p < npair - 1)
                        def _():
                            drain_idx(sx0, dx0)
                            fire2(sx0, 0, rbufs[nxt], sems[nxt])
                    drain2(rbufs[cur], sems[cur])
                    scat2(db, j0, rbufs[cur])
                return 0
            lax.fori_loop(0, npair, pair, 0)

        @pl.when(cid == 0)
        def _():
            run(u0_ref)
        @pl.when(cid == 1)
        def _():
            run(u1_ref)

        plsc.subcore_barrier()
        w0 = sid * zrows
        @pl.when(sid < NS - 1)
        def _():
            pltpu.sync_copy(acc.at[pl.ds(pl.multiple_of(w0, 8), zrows)],
                            out_ref.at[cid, pl.ds(pl.multiple_of(w0, 8), zrows)])
        @pl.when(sid == NS - 1)
        def _():
            pltpu.sync_copy(acc.at[pl.ds(N - zrows, zrows)],
                            out_ref.at[cid, pl.ds(N - zrows, zrows)])

    return k(src_rows, dst_rows, u0, u1, zeros32)


def _sc_segmax(h, batch):
    per_w = 1568
    cs_sz = 112
    nchunks = per_w // cs_sz

    @functools.partial(
        pl.kernel,
        out_type=jax.ShapeDtypeStruct((NC * NS, G, H), jnp.float32),
        mesh=_mesh(),
        compiler_params=pltpu.CompilerParams(use_tc_tiling_on_sc=False),
        scratch_types=[
            pltpu.VMEM((cs_sz, H), jnp.float32),
            pltpu.VMEM((cs_sz + 16,), jnp.int32),
            pltpu.VMEM((G, H), jnp.float32),
        ],
    )
    def k(h_ref, b_ref, out_ref, hbuf, bbuf, table):
        cid = lax.axis_index("c")
        sid = lax.axis_index("s")
        wid = sid * NC + cid
        neg = jnp.full((16,), -jnp.inf, jnp.float32)

        def init_row(i, _):
            for kk in range(H // 16):
                table[i, pl.ds(kk * 16, 16)] = neg
            return 0
        lax.fori_loop(0, G, init_row, 0)

        start = wid * per_w

        def chunk(j, _):
            cs = jnp.minimum(start + j * cs_sz, N - cs_sz)
            cs = pl.multiple_of(cs, 8)
            pltpu.sync_copy(b_ref.at[pl.ds(cs, cs_sz)], bbuf.at[pl.ds(0, cs_sz)])
            pltpu.sync_copy(h_ref.at[pl.ds(cs, cs_sz)], hbuf)

            def node(i, _):
                g = bbuf[pl.ds(i, 16)][0]
                for kk in range(H // 16):
                    t = table[g, pl.ds(kk * 16, 16)]
                    r = hbuf[i, pl.ds(kk * 16, 16)]
                    table[g, pl.ds(kk * 16, 16)] = jnp.maximum(t, r)
                return 0
            lax.fori_loop(0, cs_sz, node, 0)
            return 0
        lax.fori_loop(0, nchunks, chunk, 0)

        pltpu.sync_copy(table, out_ref.at[wid])

    return k(h, batch)


def _tc_pre1(x, cnt, W1):
    def body(x_ref, cnt_ref, w_ref, u0_ref, u1_ref, dinv_ref):
        deg = cnt_ref[0, :, 0:1] + cnt_ref[1, :, 0:1] + 1.0
        dinv = lax.rsqrt(deg)
        u = jnp.dot(x_ref[...], w_ref[...],
                    preferred_element_type=jnp.float32) * dinv
        u0_ref[...] = u[:, :32]
        u1_ref[...] = u[:, 32:]
        dinv_ref[...] = dinv

    return pl.pallas_call(
        body,
        grid=(NB,),
        in_specs=[
            pl.BlockSpec((BLK, H), lambda i: (i, 0)),
            pl.BlockSpec((2, BLK, 8), lambda i: (0, i, 0)),
            pl.BlockSpec((H, H), lambda i: (0, 0)),
        ],
        out_specs=[
            pl.BlockSpec((BLK, 32), lambda i: (i, 0)),
            pl.BlockSpec((BLK, 32), lambda i: (i, 0)),
            pl.BlockSpec((BLK, 1), lambda i: (i, 0)),
        ],
        out_shape=[
            jax.ShapeDtypeStruct((N, 32), jnp.float32),
            jax.ShapeDtypeStruct((N, 32), jnp.float32),
            jax.ShapeDtypeStruct((N, 1), jnp.float32),
        ],
    )(x, cnt, W1)


def _tc_post(acc, u0, u1, dinv, b):
    def body(acc_ref, u0_ref, u1_ref, dinv_ref, b_ref, y_ref, st_ref):
        i = pl.program_id(0)
        a = jnp.concatenate([acc_ref[0], acc_ref[1]], axis=1)
        u = jnp.concatenate([u0_ref[...], u1_ref[...]], axis=1)
        y = (a + u) * dinv_ref[...] + b_ref[...]
        y_ref[...] = y
        s0 = jnp.sum(y, axis=0, keepdims=True)
        s1 = jnp.sum(y * y, axis=0, keepdims=True)
        st = jnp.concatenate([s0, s1], axis=0)
        @pl.when(i == 0)
        def _():
            st_ref[...] = st
        @pl.when(i > 0)
        def _():
            st_ref[...] += st

    return pl.pallas_call(
        body,
        grid=(NB,),
        in_specs=[
            pl.BlockSpec((2, BLK, 32), lambda i: (0, i, 0)),
            pl.BlockSpec((BLK, 32), lambda i: (i, 0)),
            pl.BlockSpec((BLK, 32), lambda i: (i, 0)),
            pl.BlockSpec((BLK, 1), lambda i: (i, 0)),
            pl.BlockSpec((1, H), lambda i: (0, 0)),
        ],
        out_specs=[
            pl.BlockSpec((BLK, H), lambda i: (i, 0)),
            pl.BlockSpec((2, H), lambda i: (0, 0)),
        ],
        out_shape=[
            jax.ShapeDtypeStruct((N, H), jnp.float32),
            jax.ShapeDtypeStruct((2, H), jnp.float32),
        ],
    )(acc, u0, u1, dinv, b)


def _tc_pre(y, st, gam, bet, W, dinv):
    def body(y_ref, st_ref, g_ref, be_ref, w_ref, dinv_ref, u0_ref, u1_ref):
        m = st_ref[0:1, :] * (1.0 / N)
        v = st_ref[1:2, :] * (1.0 / N) - m * m
        hh = jax.nn.relu((y_ref[...] - m) * lax.rsqrt(v + 1e-5)
                         * g_ref[...] + be_ref[...])
        u = jnp.dot(hh, w_ref[...],
                    preferred_element_type=jnp.float32) * dinv_ref[...]
        u0_ref[...] = u[:, :32]
        u1_ref[...] = u[:, 32:]

    return pl.pallas_call(
        body,
        grid=(NB,),
        in_specs=[
            pl.BlockSpec((BLK, H), lambda i: (i, 0)),
            pl.BlockSpec((2, H), lambda i: (0, 0)),
            pl.BlockSpec((1, H), lambda i: (0, 0)),
            pl.BlockSpec((1, H), lambda i: (0, 0)),
            pl.BlockSpec((H, H), lambda i: (0, 0)),
            pl.BlockSpec((BLK, 1), lambda i: (i, 0)),
        ],
        out_specs=[
            pl.BlockSpec((BLK, 32), lambda i: (i, 0)),
            pl.BlockSpec((BLK, 32), lambda i: (i, 0)),
        ],
        out_shape=[
            jax.ShapeDtypeStruct((N, 32), jnp.float32),
            jax.ShapeDtypeStruct((N, 32), jnp.float32),
        ],
    )(y, st, gam, bet, W, dinv)


def _tc_pool(y, st, gam, bet, batch2d):
    def body(y_ref, st_ref, g_ref, be_ref, b_ref, h_ref, gs_ref, gc_ref):
        i = pl.program_id(0)
        m = st_ref[0:1, :] * (1.0 / N)
        v = st_ref[1:2, :] * (1.0 / N) - m * m
        hh = jax.nn.relu((y_ref[...] - m) * lax.rsqrt(v + 1e-5)
                         * g_ref[...] + be_ref[...])
        h_ref[...] = hh
        ids = lax.broadcasted_iota(jnp.int32, (G, BLK), 0)
        oh = (ids == b_ref[0]).astype(jnp.float32)
        gs = jnp.dot(oh, hh, preferred_element_type=jnp.float32)
        gc = jnp.sum(oh, axis=1, keepdims=True)
        @pl.when(i == 0)
        def _():
            gs_ref[...] = gs
            gc_ref[...] = gc
        @pl.when(i > 0)
        def _():
            gs_ref[...] += gs
            gc_ref[...] += gc

    return pl.pallas_call(
        body,
        grid=(NB,),
        in_specs=[
            pl.BlockSpec((BLK, H), lambda i: (i, 0)),
            pl.BlockSpec((2, H), lambda i: (0, 0)),
            pl.BlockSpec((1, H), lambda i: (0, 0)),
            pl.BlockSpec((1, H), lambda i: (0, 0)),
            pl.BlockSpec((1, 1, BLK), lambda i: (i, 0, 0)),
        ],
        out_specs=[
            pl.BlockSpec((BLK, H), lambda i: (i, 0)),
            pl.BlockSpec((G, H), lambda i: (0, 0)),
            pl.BlockSpec((G, 1), lambda i: (0, 0)),
        ],
        out_shape=[
            jax.ShapeDtypeStruct((N, H), jnp.float32),
            jax.ShapeDtypeStruct((G, H), jnp.float32),
            jax.ShapeDtypeStruct((G, 1), jnp.float32),
        ],
    )(y, st, gam, bet, batch2d)


def _tc_head(gs, gc, maxp, LW1, Lb1, LW2, Lb2, LW3, Lb3):
    def body(gs_ref, gc_ref, mp_ref, w1_ref, b1_ref, w2_ref, b2_ref,
             w3_ref, b3_ref, out_ref):
        mx = jnp.max(mp_ref[...], axis=0)
        mean = gs_ref[...] / jnp.maximum(gc_ref[...], 1.0)
        z = jnp.concatenate([mean, mx], axis=1)
        z = jax.nn.relu(jnp.dot(z, w1_ref[...],
                                preferred_element_type=jnp.float32) + b1_ref[...])
        z = jax.nn.relu(jnp.dot(z, w2_ref[...],
                                preferred_element_type=jnp.float32) + b2_ref[...])
        out_ref[...] = jnp.dot(z, w3_ref[...],
                               preferred_element_type=jnp.float32) + b3_ref[...]

    return pl.pallas_call(
        body,
        out_shape=jax.ShapeDtypeStruct((G, NCLS), jnp.float32),
    )(gs, gc, maxp, LW1, Lb1, LW2, Lb2, LW3, Lb3)


def kernel(x, edge_index, batch, W1, b1, W2, b2, W3, b3, g1, be1, g2, be2,
           g3, be3, LW1, Lb1, LW2, Lb2, LW3, Lb3):
    src = edge_index[0]
    dst = edge_index[1]
    pad = EPAD - E
    src_rows = jnp.concatenate(
        [src, jnp.zeros((pad,), jnp.int32)]).reshape(EROWS, 128)
    dst_rows = jnp.concatenate(
        [dst, jnp.full((pad,), N, jnp.int32)]).reshape(EROWS, 128)
    ones8 = jnp.zeros((128, 8), jnp.float32).at[:, 0].set(1.0)
    zeros8 = jnp.zeros((128, 8), jnp.float32)
    zeros32 = jnp.zeros((128, 32), jnp.float32)
    batch2d = batch.reshape(NB, 1, BLK)

    cnt = _sc_degree(dst_rows, ones8, zeros8)
    u0, u1, dinv = _tc_pre1(x, cnt, W1)

    acc = _sc_aggregate(src_rows, dst_rows, u0, u1, zeros32)
    y, st = _tc_post(acc, u0, u1, dinv, b1.reshape(1, H))

    u0, u1 = _tc_pre(y, st, g1.reshape(1, H), be1.reshape(1, H), W2, dinv)
    acc = _sc_aggregate(src_rows, dst_rows, u0, u1, zeros32)
    y, st = _tc_post(acc, u0, u1, dinv, b2.reshape(1, H))

    u0, u1 = _tc_pre(y, st, g2.reshape(1, H), be2.reshape(1, H), W3, dinv)
    acc = _sc_aggregate(src_rows, dst_rows, u0, u1, zeros32)
    y, st = _tc_post(acc, u0, u1, dinv, b3.reshape(1, H))

    h, gs, gc = _tc_pool(y, st, g3.reshape(1, H), be3.reshape(1, H), batch2d)
    maxp = _sc_segmax(h, batch)

    return _tc_head(gs, gc, maxp, LW1, Lb1.reshape(1, H),
                    LW2, Lb2.reshape(1, H // 2), LW3, Lb3.reshape(1, NCLS))
